# baseline (device time: 55416 ns/iter reference)
import jax
import jax.numpy as jnp
from jax import lax
from jax.experimental import pallas as pl
from jax.experimental.pallas import tpu as pltpu

N_DEV = 4
M = 1536
N = 1536
K_PER = 768
HALF = M // 2
Q = M // 4


def _gelu(z):
    return 0.5 * z * (1.0 + jnp.tanh(0.7978845608 * (z + 0.044715 * z * z * z)))


def kernel(A, B):
    def body(a_hbm, b_hbm, out_ref, acc_ref, vout, vA, vB,
             sA1, rA1, sB1, rB1, sA2, rA2, sB2, rB2,
             gA, gB, oA, oB, send_sems, recv_sems, copy_sems, in_sems):
        in_a = pltpu.make_async_copy(a_hbm, vA, in_sems.at[0])
        in_bl = pltpu.make_async_copy(
            b_hbm.at[:, pl.ds(0, HALF)], vB.at[:, pl.ds(0, HALF)],
            in_sems.at[1])
        in_br = pltpu.make_async_copy(
            b_hbm.at[:, pl.ds(HALF, HALF)], vB.at[:, pl.ds(HALF, HALF)],
            in_sems.at[2])
        in_a.start()
        in_bl.start()
        in_br.start()
        i = lax.axis_index("i")
        p1 = jnp.bitwise_xor(i, 1)
        p2 = 3 - i

        barrier_sem = pltpu.get_barrier_semaphore()
        for nbr in [p1, p2]:
            pl.semaphore_signal(
                barrier_sem, inc=1,
                device_id=(nbr,), device_id_type=pl.DeviceIdType.MESH,
            )
        pl.semaphore_wait(barrier_sem, 2)

        keptA = jnp.where((i == 0) | (i == 3), 0, HALF)
        sendA = HALF - keptA
        qlA = jnp.where(i >= 2, 1, 0)
        ownA = keptA + qlA * Q
        othA = keptA + (1 - qlA) * Q

        keptB = jnp.where(i <= 1, 0, HALF)
        sendB = HALF - keptB
        qlB = jnp.where(i % 2 == 1, 1, 0)
        ownB = keptB + qlB * Q
        othB = keptB + (1 - qlB) * Q

        bf16 = jnp.bfloat16
        f32 = jnp.float32

        def xfer(src, dst, sem_idx, dev):
            return pltpu.make_async_remote_copy(
                src_ref=src, dst_ref=dst,
                send_sem=send_sems.at[sem_idx], recv_sem=recv_sems.at[sem_idx],
                device_id=(dev,), device_id_type=pl.DeviceIdType.MESH,
            )

        def hblock(row_start, col_start):
            return jnp.dot(
                vA[pl.ds(row_start, HALF), :].astype(bf16),
                vB[:, pl.ds(col_start, Q)].astype(bf16),
                preferred_element_type=f32,
            )

        def mk(bfly, s):
            if bfly == "A":
                col_g = s * Q
                kept, send, ql, own, oth = keptA, sendA, qlA, ownA, othA
                s1, r1, s2, r2, g, o = sA1, rA1, sA2, rA2, gA, oA
                d1, d2 = p1, p2
                ob4a, ob4b = ql, 1 - ql
            else:
                col_g = HALF + s * Q
                kept, send, ql, own, oth = keptB, sendB, qlB, ownB, othB
                s1, r1, s2, r2, g, o = sB1, rB1, sB2, rB2, gB, oB
                d1, d2 = p2, p1
                ob4a, ob4b = 1 - ql, ql
            col_b = s * Q
            base = {"A": 0, "B": 1}[bfly] + 2 * s
            return dict(
                col_g=col_g, col_b=col_b, kept=kept, send=send, ql=ql,
                own=own, oth=oth, s1=s1, r1=r1, s2=s2, r2=r2, g=g, o=o,
                d1=d1, d2=d2, ob4a=ob4a, ob4b=ob4b, base=base,
            )

        ctxs = {(b, s): mk(b, s) for b in ("A", "B") for s in (0, 1)}

        def sem(c, phase):
            return c["base"] + 4 * phase

        out_copies = []

        def flush(c, row_start):
            cp = pltpu.make_async_copy(
                vout.at[pl.ds(row_start, Q), pl.ds(c["col_g"], Q)],
                out_ref.at[pl.ds(row_start, Q), pl.ds(c["col_g"], Q)],
                copy_sems.at[len(out_copies)],
            )
            cp.start()
            out_copies.append(cp)

        def compute_send(c):
            c["s1"][:, pl.ds(c["col_b"], Q)] = (
                hblock(c["send"], c["col_g"]).astype(bf16)
            )

        def start_p1(c):
            r = xfer(c["s1"].at[:, pl.ds(c["col_b"], Q)],
                     c["r1"].at[:, pl.ds(c["col_b"], Q)], sem(c, 0), c["d1"])
            r.start()
            return r

        def compute_kept(c):
            acc_ref[pl.ds(c["kept"], HALF), pl.ds(c["col_g"], Q)] = (
                hblock(c["kept"], c["col_g"])
            )

        def stage_start_p2(c):
            c["s2"][:, pl.ds(c["col_b"], Q)] = (
                acc_ref[pl.ds(c["oth"], Q), pl.ds(c["col_g"], Q)]
                + c["r1"][pl.ds((1 - c["ql"]) * Q, Q),
                          pl.ds(c["col_b"], Q)].astype(f32)
            ).astype(bf16)
            r = xfer(c["s2"].at[:, pl.ds(c["col_b"], Q)],
                     c["r2"].at[:, pl.ds(c["col_b"], Q)], sem(c, 1), c["d2"])
            r.start()
            return r

        def gelu_start_p3_p4a(c):
            gz = _gelu(
                acc_ref[pl.ds(c["own"], Q), pl.ds(c["col_g"], Q)]
                + c["r1"][pl.ds(c["ql"] * Q, Q),
                          pl.ds(c["col_b"], Q)].astype(f32)
                + c["r2"][:, pl.ds(c["col_b"], Q)].astype(f32)
            )
            vout[pl.ds(c["own"], Q), pl.ds(c["col_g"], Q)] = gz
            c["g"][pl.ds(c["ql"] * Q, Q), pl.ds(c["col_b"], Q)] = gz.astype(bf16)
            r3 = xfer(c["g"].at[pl.ds(c["ql"] * Q, Q), pl.ds(c["col_b"], Q)],
                      c["g"].at[pl.ds(c["ql"] * Q, Q), pl.ds(c["col_b"], Q)],
                      sem(c, 2), c["d2"])
            r4a = xfer(c["g"].at[pl.ds(c["ql"] * Q, Q), pl.ds(c["col_b"], Q)],
                       c["o"].at[pl.ds(c["ql"] * Q, Q), pl.ds(c["col_b"], Q)],
                       sem(c, 3), c["d1"])
            r3.start()
            r4a.start()
            flush(c, c["own"])
            return r3, r4a

        def start_p4b_upcast_p3(c):
            r4b = xfer(
                c["g"].at[pl.ds((1 - c["ql"]) * Q, Q), pl.ds(c["col_b"], Q)],
                c["o"].at[pl.ds((1 - c["ql"]) * Q, Q), pl.ds(c["col_b"], Q)],
                sem(c, 4), c["d1"])
            r4b.start()
            vout[pl.ds(c["oth"], Q), pl.ds(c["col_g"], Q)] = (
                c["g"][pl.ds((1 - c["ql"]) * Q, Q),
                       pl.ds(c["col_b"], Q)].astype(f32)
            )
            flush(c, c["oth"])
            return r4b

        def upcast_p4(c, slot):
            vout[pl.ds(c["send"] + slot * Q, Q), pl.ds(c["col_g"], Q)] = (
                c["o"][pl.ds(slot * Q, Q), pl.ds(c["col_b"], Q)].astype(f32)
            )
            flush(c, c["send"] + slot * Q)

        A0, B0, A1, B1 = ctxs["A", 0], ctxs["B", 0], ctxs["A", 1], ctxs["B", 1]
        order = [A0, B0, A1, B1]

        in_a.wait()
        in_bl.wait()
        in_br.wait()

        p1s = {}
        for c in order:
            compute_send(c)
            p1s[id(c)] = start_p1(c)

        for c in order:
            compute_kept(c)

        p2s = {}
        for c in order:
            p1s[id(c)].wait()
            p2s[id(c)] = stage_start_p2(c)

        p34 = {}
        for c in order:
            p2s[id(c)].wait()
            p34[id(c)] = gelu_start_p3_p4a(c)

        p4bs = {}
        for c in order:
            p34[id(c)][0].wait()
            p4bs[id(c)] = start_p4b_upcast_p3(c)

        for c in order:
            p34[id(c)][1].wait()
            upcast_p4(c, c["ob4a"])
        for c in order:
            p4bs[id(c)].wait()
            upcast_p4(c, c["ob4b"])

        for cp in out_copies:
            cp.wait()

    bf16 = jnp.bfloat16
    return pl.pallas_call(
        body,
        out_shape=jax.ShapeDtypeStruct((M, N), jnp.float32),
        in_specs=[
            pl.BlockSpec(memory_space=pltpu.MemorySpace.HBM),
            pl.BlockSpec(memory_space=pltpu.MemorySpace.HBM),
        ],
        out_specs=pl.BlockSpec(memory_space=pltpu.MemorySpace.HBM),
        scratch_shapes=[
            pltpu.VMEM((M, N), jnp.float32),
            pltpu.VMEM((M, N), jnp.float32),
            pltpu.VMEM((M, K_PER), jnp.float32),
            pltpu.VMEM((K_PER, N), jnp.float32),
            pltpu.VMEM((HALF, HALF), bf16),
            pltpu.VMEM((HALF, HALF), bf16),
            pltpu.VMEM((HALF, HALF), bf16),
            pltpu.VMEM((HALF, HALF), bf16),
            pltpu.VMEM((Q, HALF), bf16),
            pltpu.VMEM((Q, HALF), bf16),
            pltpu.VMEM((Q, HALF), bf16),
            pltpu.VMEM((Q, HALF), bf16),
            pltpu.VMEM((HALF, HALF), bf16),
            pltpu.VMEM((HALF, HALF), bf16),
            pltpu.VMEM((HALF, HALF), bf16),
            pltpu.VMEM((HALF, HALF), bf16),
            pltpu.SemaphoreType.DMA((20,)),
            pltpu.SemaphoreType.DMA((20,)),
            pltpu.SemaphoreType.DMA((16,)),
            pltpu.SemaphoreType.DMA((3,)),
        ],
        compiler_params=pltpu.CompilerParams(
            collective_id=0, vmem_limit_bytes=64 * 1024 * 1024,
        ),
    )(A, B)


# device time: 51999 ns/iter; 1.0657x vs baseline; 1.0657x over previous
import jax
import jax.numpy as jnp
from jax import lax
from jax.experimental import pallas as pl
from jax.experimental.pallas import tpu as pltpu

N_DEV = 4
M = 1536
N = 1536
HALF = M // 2
Q = M // 4


def _gelu(z):
    return 0.5 * z * (1.0 + jnp.tanh(0.7978845608 * (z + 0.044715 * z * z * z)))


def kernel(A, B):
    def body(a_ref, b_ref, out_ref, acc_ref,
             sA1, rA1, sB1, rB1, sA2, rA2, sB2, rB2,
             gA, gB, oA, oB, send_sems, recv_sems):
        i = lax.axis_index("i")
        p1 = jnp.bitwise_xor(i, 1)
        p2 = 3 - i

        barrier_sem = pltpu.get_barrier_semaphore()
        for nbr in [p1, p2]:
            pl.semaphore_signal(
                barrier_sem, inc=1,
                device_id=(nbr,), device_id_type=pl.DeviceIdType.MESH,
            )
        pl.semaphore_wait(barrier_sem, 2)

        keptA = jnp.where((i == 0) | (i == 3), 0, HALF)
        sendA = HALF - keptA
        qlA = jnp.where(i >= 2, 1, 0)
        ownA = keptA + qlA * Q
        othA = keptA + (1 - qlA) * Q

        keptB = jnp.where(i <= 1, 0, HALF)
        sendB = HALF - keptB
        qlB = jnp.where(i % 2 == 1, 1, 0)
        ownB = keptB + qlB * Q
        othB = keptB + (1 - qlB) * Q

        bf16 = jnp.bfloat16
        f32 = jnp.float32

        def xfer(src, dst, sem_idx, dev):
            return pltpu.make_async_remote_copy(
                src_ref=src, dst_ref=dst,
                send_sem=send_sems.at[sem_idx], recv_sem=recv_sems.at[sem_idx],
                device_id=(dev,), device_id_type=pl.DeviceIdType.MESH,
            )

        def hblock(row_start, col_start):
            return jnp.dot(
                a_ref[pl.ds(row_start, HALF), :].astype(bf16),
                b_ref[:, pl.ds(col_start, Q)].astype(bf16),
                preferred_element_type=f32,
            )

        def mk(bfly, s):
            if bfly == "A":
                col_g = s * Q
                kept, send, ql, own, oth = keptA, sendA, qlA, ownA, othA
                s1, r1, s2, r2, g, o = sA1, rA1, sA2, rA2, gA, oA
                d1, d2 = p1, p2
                ob4a, ob4b = ql, 1 - ql
            else:
                col_g = HALF + s * Q
                kept, send, ql, own, oth = keptB, sendB, qlB, ownB, othB
                s1, r1, s2, r2, g, o = sB1, rB1, sB2, rB2, gB, oB
                d1, d2 = p2, p1
                ob4a, ob4b = 1 - ql, ql
            col_b = s * Q
            base = {"A": 0, "B": 1}[bfly] + 2 * s
            return dict(
                col_g=col_g, col_b=col_b, kept=kept, send=send, ql=ql,
                own=own, oth=oth, s1=s1, r1=r1, s2=s2, r2=r2, g=g, o=o,
                d1=d1, d2=d2, ob4a=ob4a, ob4b=ob4b, base=base,
            )

        ctxs = {(b, s): mk(b, s) for b in ("A", "B") for s in (0, 1)}

        def sem(c, phase):
            return c["base"] + 4 * phase

        def compute_send(c):
            c["s1"][:, pl.ds(c["col_b"], Q)] = (
                hblock(c["send"], c["col_g"]).astype(bf16)
            )

        def start_p1(c):
            r = xfer(c["s1"].at[:, pl.ds(c["col_b"], Q)],
                     c["r1"].at[:, pl.ds(c["col_b"], Q)], sem(c, 0), c["d1"])
            r.start()
            return r

        def compute_kept(c):
            acc_ref[pl.ds(c["kept"], HALF), pl.ds(c["col_g"], Q)] = (
                hblock(c["kept"], c["col_g"])
            )

        def stage_start_p2(c):
            c["s2"][:, pl.ds(c["col_b"], Q)] = (
                acc_ref[pl.ds(c["oth"], Q), pl.ds(c["col_g"], Q)]
                + c["r1"][pl.ds((1 - c["ql"]) * Q, Q),
                          pl.ds(c["col_b"], Q)].astype(f32)
            ).astype(bf16)
            r = xfer(c["s2"].at[:, pl.ds(c["col_b"], Q)],
                     c["r2"].at[:, pl.ds(c["col_b"], Q)], sem(c, 1), c["d2"])
            r.start()
            return r

        def gelu_start_p3_p4a(c):
            gz = _gelu(
                acc_ref[pl.ds(c["own"], Q), pl.ds(c["col_g"], Q)]
                + c["r1"][pl.ds(c["ql"] * Q, Q),
                          pl.ds(c["col_b"], Q)].astype(f32)
                + c["r2"][:, pl.ds(c["col_b"], Q)].astype(f32)
            )
            out_ref[pl.ds(c["own"], Q), pl.ds(c["col_g"], Q)] = gz
            c["g"][pl.ds(c["ql"] * Q, Q), pl.ds(c["col_b"], Q)] = gz.astype(bf16)
            r3 = xfer(c["g"].at[pl.ds(c["ql"] * Q, Q), pl.ds(c["col_b"], Q)],
                      c["g"].at[pl.ds(c["ql"] * Q, Q), pl.ds(c["col_b"], Q)],
                      sem(c, 2), c["d2"])
            r4a = xfer(c["g"].at[pl.ds(c["ql"] * Q, Q), pl.ds(c["col_b"], Q)],
                       c["o"].at[pl.ds(c["ql"] * Q, Q), pl.ds(c["col_b"], Q)],
                       sem(c, 3), c["d1"])
            r3.start()
            r4a.start()
            return r3, r4a

        def start_p4b_upcast_p3(c):
            r4b = xfer(
                c["g"].at[pl.ds((1 - c["ql"]) * Q, Q), pl.ds(c["col_b"], Q)],
                c["o"].at[pl.ds((1 - c["ql"]) * Q, Q), pl.ds(c["col_b"], Q)],
                sem(c, 4), c["d1"])
            r4b.start()
            out_ref[pl.ds(c["oth"], Q), pl.ds(c["col_g"], Q)] = (
                c["g"][pl.ds((1 - c["ql"]) * Q, Q),
                       pl.ds(c["col_b"], Q)].astype(f32)
            )
            return r4b

        def upcast_p4(c, slot):
            out_ref[pl.ds(c["send"] + slot * Q, Q), pl.ds(c["col_g"], Q)] = (
                c["o"][pl.ds(slot * Q, Q), pl.ds(c["col_b"], Q)].astype(f32)
            )

        A0, B0, A1, B1 = ctxs["A", 0], ctxs["B", 0], ctxs["A", 1], ctxs["B", 1]
        order = [A0, B0, A1, B1]

        p1s = {}
        for c in order:
            compute_send(c)
            p1s[id(c)] = start_p1(c)

        for c in order:
            compute_kept(c)

        p2s = {}
        for c in order:
            p1s[id(c)].wait()
            p2s[id(c)] = stage_start_p2(c)

        p34 = {}
        for c in order:
            p2s[id(c)].wait()
            p34[id(c)] = gelu_start_p3_p4a(c)

        p4bs = {}
        for c in order:
            p34[id(c)][0].wait()
            p4bs[id(c)] = start_p4b_upcast_p3(c)

        for c in order:
            p34[id(c)][1].wait()
            upcast_p4(c, c["ob4a"])
        for c in order:
            p4bs[id(c)].wait()
            upcast_p4(c, c["ob4b"])

    bf16 = jnp.bfloat16
    return pl.pallas_call(
        body,
        out_shape=jax.ShapeDtypeStruct((M, N), jnp.float32),
        in_specs=[
            pl.BlockSpec(memory_space=pltpu.VMEM),
            pl.BlockSpec(memory_space=pltpu.VMEM),
        ],
        out_specs=pl.BlockSpec(memory_space=pltpu.VMEM),
        scratch_shapes=[
            pltpu.VMEM((M, N), jnp.float32),
            pltpu.VMEM((HALF, HALF), bf16),
            pltpu.VMEM((HALF, HALF), bf16),
            pltpu.VMEM((HALF, HALF), bf16),
            pltpu.VMEM((HALF, HALF), bf16),
            pltpu.VMEM((Q, HALF), bf16),
            pltpu.VMEM((Q, HALF), bf16),
            pltpu.VMEM((Q, HALF), bf16),
            pltpu.VMEM((Q, HALF), bf16),
            pltpu.VMEM((HALF, HALF), bf16),
            pltpu.VMEM((HALF, HALF), bf16),
            pltpu.VMEM((HALF, HALF), bf16),
            pltpu.VMEM((HALF, HALF), bf16),
            pltpu.SemaphoreType.DMA((20,)),
            pltpu.SemaphoreType.DMA((20,)),
        ],
        compiler_params=pltpu.CompilerParams(collective_id=0),
    )(A, B)
